# Initial kernel scaffold; baseline (speedup 1.0000x reference)
import jax
import jax.numpy as jnp
from jax.experimental import pallas as pl


def kernel(x, edge_index, edge_attr, beta, eps):
    raise NotImplementedError("write your pallas kernel here")


# speedup vs baseline: 9.9137x; 9.9137x over previous
import functools

import jax
import jax.numpy as jnp
from jax import lax
from jax.experimental import pallas as pl
from jax.experimental.pallas import tpu as pltpu
from jax.experimental.pallas import tpu_sc as plsc

N = 10000
D = 128
E = 320000

NC = 2
NS = 16
L = 16
CH = 128

CHUNKS_BC = 79
E_W = CHUNKS_BC * CH
E_PAD = NC * NS * E_W
E_HALF = NS * E_W
CHUNKS_A = E_PAD // (NS * CH)

NPAD = 10240
SEG_W = NPAD // NS
ROWS_W = NPAD // NS


def _normalize_body(x_ref, o_ref):
    x = x_ref[...]
    s = jnp.sum(x * x, axis=1, keepdims=True)
    o_ref[...] = x * lax.rsqrt(jnp.maximum(s, 1e-24))


def _final_body(xn_ref, p0_ref, p1_ref, as_ref, be_ref, ep_ref, o_ref):
    b = be_ref[0]
    ep = ep_ref[0]
    eb = jnp.exp(b)
    at = as_ref[0, :] + as_ref[1, :] + eb
    inv = 1.0 / at
    o_ref[...] = ((1.0 + ep + eb * inv)[:, None] * xn_ref[...]
                  + (p0_ref[...] + p1_ref[...]) * inv[:, None])


_TCB = 1280


def _sc_body(row_hbm, col_hbm, ea_hbm, beta_hbm, xn_hbm,
             out_hbm, asum_hbm,
             row_v, col_v, ea_v, a_v, rows_v, bscale_v, z_v, beta_v,
             sq_sp, asum_sp, acc_sp, sem):
    c = lax.axis_index("c")
    s = lax.axis_index("s")

    zero16 = jnp.zeros((L,), jnp.float32)
    for j in range(SEG_W // L):
        z_v[pl.ds(j * L, L)] = zero16

    @pl.loop(0, CH)
    def _zero_rows(i):
        for j in range(D // L):
            rows_v[i, pl.ds(j * L, L)] = zero16

    pltpu.sync_copy(z_v, sq_sp.at[pl.ds(s * SEG_W, SEG_W)])
    pltpu.sync_copy(z_v, asum_sp.at[pl.ds(s * SEG_W, SEG_W)])
    for t in range(ROWS_W // CH):
        pltpu.sync_copy(rows_v, acc_sp.at[pl.ds(s * ROWS_W + t * CH, CH)])
    plsc.subcore_barrier()

    base_a = s * (CHUNKS_A * CH)

    @pl.loop(0, CHUNKS_A)
    def _chunk_a(k):
        off = base_a + k * CH
        pltpu.sync_copy(row_hbm.at[pl.ds(off, CH)], row_v)
        pltpu.sync_copy(col_hbm.at[pl.ds(off, CH)], col_v)
        pltpu.sync_copy(ea_hbm.at[pl.ds(off, CH)], ea_v)
        for j in range(CH // L):
            sl = pl.ds(j * L, L)
            m = row_v[sl] != col_v[sl]
            em = jnp.where(m, ea_v[sl], 0.0)
            a_v[sl] = em * em
        pltpu.sync_copy(a_v, sq_sp.at[row_v], add=True)

    plsc.subcore_barrier()

    pltpu.sync_copy(beta_hbm, beta_v)
    b = beta_v[pl.ds(0, L)][0]
    pltpu.sync_copy(sq_sp, bscale_v)

    @pl.loop(0, NPAD // L)
    def _rsqrt(i):
        sl = pl.ds(i * L, L)
        xx = jnp.maximum(bscale_v[sl], 1e-24)
        xi = plsc.bitcast(xx, jnp.int32)
        y = plsc.bitcast(jnp.int32(0x5F3759DF) - (xi >> 1), jnp.float32)
        y = y * (1.5 - 0.5 * xx * y * y)
        y = y * (1.5 - 0.5 * xx * y * y)
        y = y * (1.5 - 0.5 * xx * y * y)
        bscale_v[sl] = y * b

    base_bc = c * E_HALF + s * E_W

    @pl.loop(0, CHUNKS_BC)
    def _chunk_bc(k):
        off = base_bc + k * CH
        pltpu.sync_copy(row_hbm.at[pl.ds(off, CH)], row_v)
        pltpu.sync_copy(col_hbm.at[pl.ds(off, CH)], col_v)
        pltpu.sync_copy(ea_hbm.at[pl.ds(off, CH)], ea_v)
        gat = pltpu.async_copy(xn_hbm.at[col_v], rows_v, sem)
        for j in range(CH // L):
            sl = pl.ds(j * L, L)
            r = row_v[sl]
            m = r != col_v[sl]
            em = jnp.where(m, ea_v[sl], 0.0)
            bs = plsc.load_gather(bscale_v, [r])
            a_v[sl] = jnp.where(m, jnp.exp(em * bs), 0.0)
        pltpu.sync_copy(a_v, asum_sp.at[row_v], add=True)
        gat.wait()

        @pl.loop(0, CH // L)
        def _scale(g):
            aw = a_v[pl.ds(g * L, L)]
            for t in range(L):
                w = aw[t]
                i = g * L + t
                for j in range(D // L):
                    sl2 = pl.ds(j * L, L)
                    rows_v[i, sl2] = rows_v[i, sl2] * w

        pltpu.sync_copy(rows_v, acc_sp.at[row_v], add=True)

    plsc.subcore_barrier()

    pltpu.sync_copy(asum_sp.at[pl.ds(s * SEG_W, SEG_W)], asum_hbm.at[c, s])
    for t in range(ROWS_W // CH):
        st = s * ROWS_W + t * CH
        pltpu.sync_copy(acc_sp.at[pl.ds(st, CH)], out_hbm.at[c, pl.ds(st, CH)])


_sc_call = functools.partial(
    pl.kernel,
    out_type=(jax.ShapeDtypeStruct((NC, NPAD, D), jnp.float32),
              jax.ShapeDtypeStruct((NC, NS, SEG_W), jnp.float32)),
    mesh=plsc.VectorSubcoreMesh(core_axis_name="c", subcore_axis_name="s",
                                num_cores=NC, num_subcores=NS),
    compiler_params=pltpu.CompilerParams(needs_layout_passes=False),
    scratch_types=[
        pltpu.VMEM((CH,), jnp.int32),
        pltpu.VMEM((CH,), jnp.int32),
        pltpu.VMEM((CH,), jnp.float32),
        pltpu.VMEM((CH,), jnp.float32),
        pltpu.VMEM((CH, D), jnp.float32),
        pltpu.VMEM((NPAD,), jnp.float32),
        pltpu.VMEM((SEG_W,), jnp.float32),
        pltpu.VMEM((L,), jnp.float32),
        pltpu.VMEM_SHARED((NPAD,), jnp.float32),
        pltpu.VMEM_SHARED((NPAD,), jnp.float32),
        pltpu.VMEM_SHARED((NPAD, D), jnp.float32),
        pltpu.SemaphoreType.DMA,
    ],
)(_sc_body)


def kernel(x, edge_index, edge_attr, beta, eps):
    pad = E_PAD - E
    row = jnp.concatenate([edge_index[0], jnp.zeros((pad,), jnp.int32)])
    col = jnp.concatenate([edge_index[1], jnp.zeros((pad,), jnp.int32)])
    ea = jnp.concatenate([edge_attr, jnp.zeros((pad,), jnp.float32)])
    beta16 = jnp.broadcast_to(beta.astype(jnp.float32), (L,))
    xpad = jnp.concatenate([x, jnp.zeros((NPAD - N, D), jnp.float32)])

    xn = pl.pallas_call(
        _normalize_body,
        grid=(NPAD // _TCB,),
        in_specs=[pl.BlockSpec((_TCB, D), lambda i: (i, 0))],
        out_specs=pl.BlockSpec((_TCB, D), lambda i: (i, 0)),
        out_shape=jax.ShapeDtypeStruct((NPAD, D), jnp.float32),
    )(xpad)

    partials, asum_parts = _sc_call(row, col, ea, beta16, xn)
    asum2 = asum_parts.reshape(NC, NPAD)

    out = pl.pallas_call(
        _final_body,
        grid=(NPAD // _TCB,),
        in_specs=[
            pl.BlockSpec((_TCB, D), lambda i: (i, 0)),
            pl.BlockSpec((_TCB, D), lambda i: (i, 0)),
            pl.BlockSpec((_TCB, D), lambda i: (i, 0)),
            pl.BlockSpec((NC, _TCB), lambda i: (0, i)),
            pl.BlockSpec(memory_space=pltpu.SMEM),
            pl.BlockSpec(memory_space=pltpu.SMEM),
        ],
        out_specs=pl.BlockSpec((_TCB, D), lambda i: (i, 0)),
        out_shape=jax.ShapeDtypeStruct((NPAD, D), jnp.float32),
    )(xn, partials[0], partials[1], asum2,
      beta.astype(jnp.float32), eps.astype(jnp.float32))
    return out[:N]


# speedup vs baseline: 15.6942x; 1.5831x over previous
import functools

import jax
import jax.numpy as jnp
from jax import lax
from jax.experimental import pallas as pl
from jax.experimental.pallas import tpu as pltpu
from jax.experimental.pallas import tpu_sc as plsc

N = 10000
D = 128
E = 320000

NC = 2
NS = 16
L = 16
CH = 128

CHUNKS_BC = 80
E_PAD = NC * NS * CHUNKS_BC * CH
NCH = E_PAD // CH
CHUNKS_A = NCH // NS
GA = 8
A_OUTER = CHUNKS_A // GA

NPAD = 10240
SEG_W = NPAD // NS
ROWS_W = NPAD // NS


def _normalize_body(x_ref, o_ref):
    x = x_ref[...]
    s = jnp.sum(x * x, axis=1, keepdims=True)
    o_ref[...] = x * lax.rsqrt(jnp.maximum(s, 1e-24))


def _final_body(xn_ref, p0_ref, p1_ref, as_ref, be_ref, ep_ref, o_ref):
    b = be_ref[0]
    ep = ep_ref[0]
    eb = jnp.exp(b)
    at = as_ref[0, :] + as_ref[1, :] + eb
    inv = 1.0 / at
    o_ref[...] = ((1.0 + ep + eb * inv)[:, None] * xn_ref[...]
                  + (p0_ref[...] + p1_ref[...]) * inv[:, None])


_TCB = 1280


def _sc_body(row_hbm, col_hbm, ea_hbm, beta_hbm, xn_hbm,
             out_hbm, asum_hbm,
             ra0, ca0, ea0s, ra1, ca1, ea1s,
             row0, col0, ea0, row1, col1, ea1, a_v, bs0, bs1,
             rows0, rows1, z_v, beta_v,
             sq_sp, asum_sp, acc_sp,
             la0, la1, sem_s, sem0, sem1, sb0, sb1):
    c = lax.axis_index("c")
    s = lax.axis_index("s")
    a_base = s * CHUNKS_A

    def _slab_load(g, rp, cp, ep2, semp):
        off = a_base + g * GA
        pltpu.async_copy(row_hbm.at[pl.ds(off, GA)], rp, semp)
        pltpu.async_copy(col_hbm.at[pl.ds(off, GA)], cp, semp)
        pltpu.async_copy(ea_hbm.at[pl.ds(off, GA)], ep2, semp)

    def _slab_wait(rp, cp, ep2, semp):
        pltpu.make_async_copy(row_hbm.at[pl.ds(0, GA)], rp, semp).wait()
        pltpu.make_async_copy(col_hbm.at[pl.ds(0, GA)], cp, semp).wait()
        pltpu.make_async_copy(ea_hbm.at[pl.ds(0, GA)], ep2, semp).wait()

    def _slab_proc(rp, cp, ep2):
        descs = []
        for u in range(GA):
            for t in range(CH // L):
                sl = pl.ds(t * L, L)
                m = rp[u, sl] != cp[u, sl]
                em = jnp.where(m, ep2[u, sl], 0.0)
                ep2[u, sl] = em * em
            descs.append(
                pltpu.async_copy(ep2.at[u], sq_sp.at[rp.at[u]], sem_s,
                                 add=True))
        for dsc in descs:
            dsc.wait()

    _slab_load(0, ra0, ca0, ea0s, la0)

    zero16 = jnp.zeros((L,), jnp.float32)
    for j in range(SEG_W // L):
        z_v[pl.ds(j * L, L)] = zero16

    @pl.loop(0, CH)
    def _zero_rows(i):
        for j in range(D // L):
            rows0[i, pl.ds(j * L, L)] = zero16

    pltpu.sync_copy(z_v, sq_sp.at[pl.ds(s * SEG_W, SEG_W)])
    pltpu.sync_copy(z_v, asum_sp.at[pl.ds(s * SEG_W, SEG_W)])
    for t in range(ROWS_W // CH):
        pltpu.sync_copy(rows0, acc_sp.at[pl.ds(s * ROWS_W + t * CH, CH)])
    plsc.subcore_barrier()

    @pl.loop(0, A_OUTER // 2)
    def _chunk_a(g2):
        g = 2 * g2
        _slab_wait(ra0, ca0, ea0s, la0)
        _slab_load(g + 1, ra1, ca1, ea1s, la1)
        _slab_proc(ra0, ca0, ea0s)
        _slab_wait(ra1, ca1, ea1s, la1)

        @pl.when(g2 < A_OUTER // 2 - 1)
        def _():
            _slab_load(g + 2, ra0, ca0, ea0s, la0)

        _slab_proc(ra1, ca1, ea1s)

    plsc.subcore_barrier()

    pltpu.sync_copy(beta_hbm, beta_v)
    b = beta_v[pl.ds(0, L)][0]
    pltpu.sync_copy(sq_sp.at[pl.ds(s * SEG_W, SEG_W)], z_v)

    @pl.loop(0, SEG_W // L)
    def _rsqrt(i):
        sl = pl.ds(i * L, L)
        xx = jnp.maximum(z_v[sl], 1e-24)
        xi = plsc.bitcast(xx, jnp.int32)
        y = plsc.bitcast(jnp.int32(0x5F3759DF) - (xi >> 1), jnp.float32)
        y = y * (1.5 - 0.5 * xx * y * y)
        y = y * (1.5 - 0.5 * xx * y * y)
        y = y * (1.5 - 0.5 * xx * y * y)
        z_v[sl] = y * b

    pltpu.sync_copy(z_v, sq_sp.at[pl.ds(s * SEG_W, SEG_W)])
    plsc.subcore_barrier()

    base_bc = c * (NS * CHUNKS_BC) + s * CHUNKS_BC

    def _load_and_gather(k_next, row_q, col_q, ea_q, rows_q, bs_q,
                         sem_q, sb_q):
        pltpu.sync_copy(row_hbm.at[k_next], row_q)
        pltpu.sync_copy(col_hbm.at[k_next], col_q)
        pltpu.sync_copy(ea_hbm.at[k_next], ea_q)
        pltpu.async_copy(xn_hbm.at[col_q], rows_q, sem_q)
        pltpu.async_copy(sq_sp.at[row_q], bs_q, sb_q)

    def _half(row_p, col_p, ea_p, rows_p, bs_p, sem_p, sb_p):
        pltpu.make_async_copy(sq_sp.at[row_p], bs_p, sb_p).wait()
        for j in range(CH // L):
            sl = pl.ds(j * L, L)
            m = row_p[sl] != col_p[sl]
            em = jnp.where(m, ea_p[sl], 0.0)
            a_v[sl] = jnp.where(m, jnp.exp(em * bs_p[sl]), 0.0)
        pltpu.sync_copy(a_v, asum_sp.at[row_p], add=True)
        pltpu.make_async_copy(xn_hbm.at[col_p], rows_p, sem_p).wait()

        @pl.loop(0, CH // L)
        def _scale(g):
            aw = a_v[pl.ds(g * L, L)]
            for t in range(L):
                w = aw[t]
                i = g * L + t
                for j in range(D // L):
                    sl2 = pl.ds(j * L, L)
                    rows_p[i, sl2] = rows_p[i, sl2] * w

        pltpu.sync_copy(rows_p, acc_sp.at[row_p], add=True)

    _load_and_gather(base_bc, row0, col0, ea0, rows0, bs0, sem0, sb0)

    @pl.loop(0, CHUNKS_BC // 2)
    def _chunk_bc(g):
        k0 = base_bc + 2 * g
        _load_and_gather(k0 + 1, row1, col1, ea1, rows1, bs1, sem1, sb1)
        _half(row0, col0, ea0, rows0, bs0, sem0, sb0)

        @pl.when(g < CHUNKS_BC // 2 - 1)
        def _():
            _load_and_gather(k0 + 2, row0, col0, ea0, rows0, bs0, sem0, sb0)

        _half(row1, col1, ea1, rows1, bs1, sem1, sb1)

    plsc.subcore_barrier()

    pltpu.sync_copy(asum_sp.at[pl.ds(s * SEG_W, SEG_W)], asum_hbm.at[c, s])
    for t in range(ROWS_W // CH):
        st = s * ROWS_W + t * CH
        pltpu.sync_copy(acc_sp.at[pl.ds(st, CH)], out_hbm.at[c, pl.ds(st, CH)])


_sc_call = functools.partial(
    pl.kernel,
    out_type=(jax.ShapeDtypeStruct((NC, NPAD, D), jnp.float32),
              jax.ShapeDtypeStruct((NC, NS, SEG_W), jnp.float32)),
    mesh=plsc.VectorSubcoreMesh(core_axis_name="c", subcore_axis_name="s",
                                num_cores=NC, num_subcores=NS),
    compiler_params=pltpu.CompilerParams(needs_layout_passes=False),
    scratch_types=[
        pltpu.VMEM((GA, CH), jnp.int32),
        pltpu.VMEM((GA, CH), jnp.int32),
        pltpu.VMEM((GA, CH), jnp.float32),
        pltpu.VMEM((GA, CH), jnp.int32),
        pltpu.VMEM((GA, CH), jnp.int32),
        pltpu.VMEM((GA, CH), jnp.float32),
        pltpu.VMEM((CH,), jnp.int32),
        pltpu.VMEM((CH,), jnp.int32),
        pltpu.VMEM((CH,), jnp.float32),
        pltpu.VMEM((CH,), jnp.int32),
        pltpu.VMEM((CH,), jnp.int32),
        pltpu.VMEM((CH,), jnp.float32),
        pltpu.VMEM((CH,), jnp.float32),
        pltpu.VMEM((CH,), jnp.float32),
        pltpu.VMEM((CH,), jnp.float32),
        pltpu.VMEM((CH, D), jnp.float32),
        pltpu.VMEM((CH, D), jnp.float32),
        pltpu.VMEM((SEG_W,), jnp.float32),
        pltpu.VMEM((L,), jnp.float32),
        pltpu.VMEM_SHARED((NPAD,), jnp.float32),
        pltpu.VMEM_SHARED((NPAD,), jnp.float32),
        pltpu.VMEM_SHARED((NPAD, D), jnp.float32),
        pltpu.SemaphoreType.DMA,
        pltpu.SemaphoreType.DMA,
        pltpu.SemaphoreType.DMA,
        pltpu.SemaphoreType.DMA,
        pltpu.SemaphoreType.DMA,
        pltpu.SemaphoreType.DMA,
        pltpu.SemaphoreType.DMA,
    ],
)(_sc_body)


def kernel(x, edge_index, edge_attr, beta, eps):
    pad = E_PAD - E
    row = jnp.concatenate([edge_index[0], jnp.zeros((pad,), jnp.int32)])
    col = jnp.concatenate([edge_index[1], jnp.zeros((pad,), jnp.int32)])
    ea = jnp.concatenate([edge_attr, jnp.zeros((pad,), jnp.float32)])
    row2d = row.reshape(NCH, CH)
    col2d = col.reshape(NCH, CH)
    ea2d = ea.reshape(NCH, CH)
    beta16 = jnp.broadcast_to(beta.astype(jnp.float32), (L,))
    xpad = jnp.concatenate([x, jnp.zeros((NPAD - N, D), jnp.float32)])

    xn = pl.pallas_call(
        _normalize_body,
        grid=(NPAD // _TCB,),
        in_specs=[pl.BlockSpec((_TCB, D), lambda i: (i, 0))],
        out_specs=pl.BlockSpec((_TCB, D), lambda i: (i, 0)),
        out_shape=jax.ShapeDtypeStruct((NPAD, D), jnp.float32),
    )(xpad)

    partials, asum_parts = _sc_call(row2d, col2d, ea2d, beta16, xn)
    asum2 = asum_parts.reshape(NC, NPAD)

    out = pl.pallas_call(
        _final_body,
        grid=(NPAD // _TCB,),
        in_specs=[
            pl.BlockSpec((_TCB, D), lambda i: (i, 0)),
            pl.BlockSpec((_TCB, D), lambda i: (i, 0)),
            pl.BlockSpec((_TCB, D), lambda i: (i, 0)),
            pl.BlockSpec((NC, _TCB), lambda i: (0, i)),
            pl.BlockSpec(memory_space=pltpu.SMEM),
            pl.BlockSpec(memory_space=pltpu.SMEM),
        ],
        out_specs=pl.BlockSpec((_TCB, D), lambda i: (i, 0)),
        out_shape=jax.ShapeDtypeStruct((NPAD, D), jnp.float32),
    )(xn, partials[0], partials[1], asum2,
      beta.astype(jnp.float32), eps.astype(jnp.float32))
    return out[:N]
